# baseline (speedup 1.0000x reference)
import functools

import jax
import jax.numpy as jnp
from jax import lax
from jax.experimental import pallas as pl
from jax.experimental.pallas import tpu as pltpu
from jax.experimental.pallas import tpu_sc as plsc

N = 10000
D = 128
C = 64
E = 320000

NC = 2
NS = 16
NW = NC * NS
EPW = E // NW
CH = 125
NCH = EPW // CH
CBASE = NW * NCH
NP = 10240
RPT = NP // NS
DW = 16
PCH = 128
RPW = NP // NW

BT = 2000
GT = N // BT

_mesh = plsc.VectorSubcoreMesh(core_axis_name="c", subcore_axis_name="s")
_sc_params = pltpu.CompilerParams(use_tc_tiling_on_sc=False)


def _memset_zero(buf, nrows, ncols):
    zv = jnp.zeros((16,), jnp.float32)

    def body(r, carry):
        for j in range(ncols // 16):
            buf[r, pl.ds(j * 16, 16)] = zv
        return carry

    lax.fori_loop(0, nrows, body, 0)


def _zero_acc(acc, zbuf, sid, width):
    _memset_zero(zbuf, PCH, width)
    for k in range(RPT // PCH):
        pltpu.sync_copy(zbuf, acc.at[pl.ds(sid * RPT + k * PCH, PCH)])


NSLOT = 4
PF = 2


def _edge_phase(u_hbm, acc, rowall, colall, rows, semg, sems):
    def _gather(s, slot):
        pltpu.async_copy(u_hbm.at[rowall.at[s]], rows.at[slot], semg[slot])

    def _wait_gather(s, slot):
        pltpu.make_async_copy(u_hbm.at[rowall.at[s]], rows.at[slot],
                              semg[slot]).wait()

    def _scatter(s, slot):
        pltpu.async_copy(rows.at[slot], acc.at[colall.at[s]], sems[slot],
                         add=True)

    def _wait_scatter(s, slot):
        pltpu.make_async_copy(rows.at[slot], acc.at[colall.at[s]],
                              sems[slot]).wait()

    for s in range(PF):
        _gather(s, s)

    def body(g, carry):
        for j in range(NSLOT):
            s = g * NSLOT + j
            _wait_gather(s, j)
            _scatter(s, j)

            @pl.when(s >= PF)
            def _():
                _wait_scatter(s - PF, (j - PF) % NSLOT)

            @pl.when(s + PF < NCH)
            def _():
                _gather(s + PF, (j + PF) % NSLOT)

        return carry

    lax.fori_loop(0, NCH // NSLOT, body, 0)
    for s in range(NCH - PF, NCH):
        _wait_scatter(s, s % NSLOT)


def _writeback(acc, out0, out1, cid, sid):
    @pl.when(cid == 0)
    def _():
        pltpu.sync_copy(acc.at[pl.ds(sid * RPT, RPT)],
                        out0.at[pl.ds(sid * RPT, RPT)])

    @pl.when(cid == 1)
    def _():
        pltpu.sync_copy(acc.at[pl.ds(sid * RPT, RPT)],
                        out1.at[pl.ds(sid * RPT, RPT)])


@functools.partial(
    pl.kernel,
    mesh=_mesh,
    compiler_params=_sc_params,
    out_type=[jax.ShapeDtypeStruct((NP, DW), jnp.float32)] * 2,
    scratch_types=[
        pltpu.VMEM((NCH, CH), jnp.int32),
        pltpu.VMEM((CH, DW), jnp.float32),
        pltpu.VMEM((PCH, DW), jnp.float32),
        pltpu.VMEM_SHARED((NP, DW), jnp.float32),
    ],
)
def _sc_deg(edges_hbm, out0, out1, colall, onesbuf, zbuf, dacc):
    cid = lax.axis_index("c")
    sid = lax.axis_index("s")
    wid = sid * NC + cid
    _zero_acc(dacc, zbuf, sid, DW)
    ov = jnp.ones((16,), jnp.float32)

    def fill(r, carry):
        onesbuf[r, pl.ds(0, 16)] = ov
        return carry

    lax.fori_loop(0, CH, fill, 0)
    pltpu.sync_copy(edges_hbm.at[pl.ds(CBASE + wid * NCH, NCH)], colall)
    plsc.subcore_barrier()

    def body(step, carry):
        pltpu.sync_copy(onesbuf, dacc.at[colall.at[step]], add=True)
        return carry

    lax.fori_loop(0, NCH, body, 0)
    plsc.subcore_barrier()
    _writeback(dacc, out0, out1, cid, sid)


@functools.partial(
    pl.kernel,
    mesh=_mesh,
    compiler_params=_sc_params,
    out_type=[jax.ShapeDtypeStruct((NP, C), jnp.float32)] * 2,
    scratch_types=[
        pltpu.VMEM((NCH, CH), jnp.int32),
        pltpu.VMEM((NCH, CH), jnp.int32),
        pltpu.VMEM((NSLOT, CH, C), jnp.float32),
        pltpu.VMEM((PCH, C), jnp.float32),
        pltpu.VMEM_SHARED((NP, C), jnp.float32),
        [pltpu.SemaphoreType.DMA] * NSLOT,
        [pltpu.SemaphoreType.DMA] * NSLOT,
    ],
)
def _sc_hop(u_hbm, edges_hbm, out0, out1,
            rowall, colall, rows, zbuf, acc, semg, sems):
    cid = lax.axis_index("c")
    sid = lax.axis_index("s")
    wid = sid * NC + cid
    _zero_acc(acc, zbuf, sid, C)
    pltpu.sync_copy(edges_hbm.at[pl.ds(wid * NCH, NCH)], rowall)
    pltpu.sync_copy(edges_hbm.at[pl.ds(CBASE + wid * NCH, NCH)], colall)
    plsc.subcore_barrier()
    _edge_phase(u_hbm, acc, rowall, colall, rows, semg, sems)
    plsc.subcore_barrier()
    _writeback(acc, out0, out1, cid, sid)


def _tc_matmul_body(x_ref, w_ref, z_ref):
    z_ref[...] = lax.dot_general(x_ref[...], w_ref[...],
                                 (((1,), (1,)), ((), ())),
                                 preferred_element_type=jnp.float32)


_tc_matmul = pl.pallas_call(
    _tc_matmul_body,
    grid=(GT,),
    in_specs=[
        pl.BlockSpec((BT, D), lambda i: (i, 0)),
        pl.BlockSpec((C, D), lambda i: (0, 0)),
    ],
    out_specs=pl.BlockSpec((BT, C), lambda i: (i, 0)),
    out_shape=jax.ShapeDtypeStruct((NP, C), jnp.float32),
)


@functools.partial(
    pl.kernel,
    mesh=_mesh,
    compiler_params=_sc_params,
    out_type=[
        jax.ShapeDtypeStruct((NP, C), jnp.float32),
        jax.ShapeDtypeStruct((NP, DW), jnp.float32),
    ],
    scratch_types=[
        pltpu.VMEM((RPW, C), jnp.float32),
        pltpu.VMEM((RPW, DW), jnp.float32),
        pltpu.VMEM((RPW, DW), jnp.float32),
        pltpu.VMEM((RPW, C), jnp.float32),
        pltpu.VMEM((RPW, DW), jnp.float32),
    ],
)
def _sc_scale(z_hbm, dg0_hbm, dg1_hbm, u0o, sclo,
              zb, d0b, d1b, u0b, sclb):
    cid = lax.axis_index("c")
    sid = lax.axis_index("s")
    wid = sid * NC + cid
    base = wid * RPW
    pltpu.sync_copy(z_hbm.at[pl.ds(base, RPW)], zb)
    pltpu.sync_copy(dg0_hbm.at[pl.ds(base, RPW)], d0b)
    pltpu.sync_copy(dg1_hbm.at[pl.ds(base, RPW)], d1b)
    lanes = lax.iota(jnp.int32, 16)

    def rowfn(r, carry):
        deg = d0b[r, pl.ds(0, 16)] + d1b[r, pl.ds(0, 16)] + 2.0
        i = lax.bitcast_convert_type(deg, jnp.int32)
        i = 0x5F3759DF - lax.shift_right_logical(i, 1)
        y = lax.bitcast_convert_type(i, jnp.float32)
        for _ in range(3):
            y = y * (1.5 - 0.5 * deg * y * y)
        d = y[0]
        v = 2.0 * d * d
        sclb[r, pl.ds(0, 16)] = jnp.where(lanes == 0, d, 0.0) + \
            jnp.where(lanes == 1, v, 0.0)
        for j in range(C // 16):
            sl = pl.ds(j * 16, 16)
            u0b[r, sl] = d * zb[r, sl]
        return carry

    lax.fori_loop(0, RPW, rowfn, 0)
    pltpu.sync_copy(u0b, u0o.at[pl.ds(base, RPW)])
    pltpu.sync_copy(sclb, sclo.at[pl.ds(base, RPW)])


@functools.partial(
    pl.kernel,
    mesh=_mesh,
    compiler_params=_sc_params,
    out_type=[jax.ShapeDtypeStruct((NP, C), jnp.float32)] * 2,
    scratch_types=[
        pltpu.VMEM((RPW, C), jnp.float32),
        pltpu.VMEM((RPW, C), jnp.float32),
        pltpu.VMEM((RPW, C), jnp.float32),
        pltpu.VMEM((RPW, DW), jnp.float32),
        pltpu.VMEM((RPW, C), jnp.float32),
        pltpu.VMEM((RPW, C), jnp.float32),
    ],
)
def _sc_comb(p0_hbm, p1_hbm, z_hbm, scl_hbm, u1o, h1o,
             p0b, p1b, zb, sclb, u1b, h1b):
    cid = lax.axis_index("c")
    sid = lax.axis_index("s")
    wid = sid * NC + cid
    base = wid * RPW
    pltpu.sync_copy(p0_hbm.at[pl.ds(base, RPW)], p0b)
    pltpu.sync_copy(p1_hbm.at[pl.ds(base, RPW)], p1b)
    pltpu.sync_copy(z_hbm.at[pl.ds(base, RPW)], zb)
    pltpu.sync_copy(scl_hbm.at[pl.ds(base, RPW)], sclb)

    def rowfn(r, carry):
        sv = sclb[r, pl.ds(0, 16)]
        d = sv[0]
        v = sv[1]
        for j in range(C // 16):
            sl = pl.ds(j * 16, 16)
            h = d * (p0b[r, sl] + p1b[r, sl]) + v * zb[r, sl]
            h1b[r, sl] = h
            u1b[r, sl] = d * h
        return carry

    lax.fori_loop(0, RPW, rowfn, 0)
    pltpu.sync_copy(u1b, u1o.at[pl.ds(base, RPW)])
    pltpu.sync_copy(h1b, h1o.at[pl.ds(base, RPW)])


def _tc_comb2_body(p0_ref, p1_ref, scl_ref, h_ref, b_ref, o_ref):
    s = scl_ref[...]
    dis = s[:, :1]
    dinv2 = s[:, 1:2]
    t = dis * (p0_ref[...] + p1_ref[...]) + dinv2 * h_ref[...] + b_ref[...]
    m = jnp.max(t, axis=1, keepdims=True)
    lse = jnp.log(jnp.sum(jnp.exp(t - m), axis=1, keepdims=True)) + m
    o_ref[...] = t - lse


_tc_comb2 = pl.pallas_call(
    _tc_comb2_body,
    grid=(GT,),
    in_specs=[
        pl.BlockSpec((BT, C), lambda i: (i, 0)),
        pl.BlockSpec((BT, C), lambda i: (i, 0)),
        pl.BlockSpec((BT, DW), lambda i: (i, 0)),
        pl.BlockSpec((BT, C), lambda i: (i, 0)),
        pl.BlockSpec((1, C), lambda i: (0, 0)),
    ],
    out_specs=pl.BlockSpec((BT, C), lambda i: (i, 0)),
    out_shape=jax.ShapeDtypeStruct((N, C), jnp.float32),
)


def kernel(x, edge_index, W, b):
    edges = edge_index.reshape(2 * NW * NCH, CH)
    dg0, dg1 = _sc_deg(edges)
    z = _tc_matmul(x, W)
    u0, scl = _sc_scale(z, dg0, dg1)
    p10, p11 = _sc_hop(u0, edges)
    u1, h1 = _sc_comb(p10, p11, z, scl)
    p20, p21 = _sc_hop(u1, edges)
    return _tc_comb2(p20, p21, scl, h1, b.reshape(1, C))


# speedup vs baseline: 1.0121x; 1.0121x over previous
import functools

import jax
import jax.numpy as jnp
from jax import lax
from jax.experimental import pallas as pl
from jax.experimental.pallas import tpu as pltpu
from jax.experimental.pallas import tpu_sc as plsc

N = 10000
D = 128
C = 64
E = 320000

NC = 2
NS = 16
NW = NC * NS
EPW = E // NW
CH = 125
NCH = EPW // CH
CBASE = NW * NCH
NP = 10240
RPT = NP // NS
DW = 16
PCH = 128
RPW = NP // NW

BT = 2000
GT = N // BT

_mesh = plsc.VectorSubcoreMesh(core_axis_name="c", subcore_axis_name="s")
_sc_params = pltpu.CompilerParams(use_tc_tiling_on_sc=False)


def _memset_zero(buf, nrows, ncols):
    zv = jnp.zeros((16,), jnp.float32)

    def body(r, carry):
        for j in range(ncols // 16):
            buf[r, pl.ds(j * 16, 16)] = zv
        return carry

    lax.fori_loop(0, nrows, body, 0)


def _zero_acc(acc, zbuf, sid, width):
    _memset_zero(zbuf, PCH, width)
    for k in range(RPT // PCH):
        pltpu.sync_copy(zbuf, acc.at[pl.ds(sid * RPT + k * PCH, PCH)])


NSLOT = 4
PF = 2


def _edge_phase(u_hbm, acc, rowall, colall, rows, semg, sems):
    def _gather(s, slot):
        pltpu.async_copy(u_hbm.at[rowall.at[s]], rows.at[slot], semg[slot])

    def _wait_gather(s, slot):
        pltpu.make_async_copy(u_hbm.at[rowall.at[s]], rows.at[slot],
                              semg[slot]).wait()

    def _scatter(s, slot):
        pltpu.async_copy(rows.at[slot], acc.at[colall.at[s]], sems[slot],
                         add=True)

    def _wait_scatter(s, slot):
        pltpu.make_async_copy(rows.at[slot], acc.at[colall.at[s]],
                              sems[slot]).wait()

    for s in range(PF):
        _gather(s, s)

    def body(g, carry):
        for j in range(NSLOT):
            s = g * NSLOT + j
            _wait_gather(s, j)
            _scatter(s, j)

            @pl.when(s >= PF)
            def _():
                _wait_scatter(s - PF, (j - PF) % NSLOT)

            @pl.when(s + PF < NCH)
            def _():
                _gather(s + PF, (j + PF) % NSLOT)

        return carry

    lax.fori_loop(0, NCH // NSLOT, body, 0)
    for s in range(NCH - PF, NCH):
        _wait_scatter(s, s % NSLOT)


def _writeback(acc, out0, out1, cid, sid):
    @pl.when(cid == 0)
    def _():
        pltpu.sync_copy(acc.at[pl.ds(sid * RPT, RPT)],
                        out0.at[pl.ds(sid * RPT, RPT)])

    @pl.when(cid == 1)
    def _():
        pltpu.sync_copy(acc.at[pl.ds(sid * RPT, RPT)],
                        out1.at[pl.ds(sid * RPT, RPT)])


@functools.partial(
    pl.kernel,
    mesh=_mesh,
    compiler_params=_sc_params,
    out_type=[jax.ShapeDtypeStruct((NP, DW), jnp.float32)] * 2,
    scratch_types=[
        pltpu.VMEM((NCH, CH), jnp.int32),
        pltpu.VMEM((CH, DW), jnp.float32),
        pltpu.VMEM((PCH, DW), jnp.float32),
        pltpu.VMEM_SHARED((NP, DW), jnp.float32),
    ],
)
def _sc_deg(edges_hbm, out0, out1, colall, onesbuf, zbuf, dacc):
    cid = lax.axis_index("c")
    sid = lax.axis_index("s")
    wid = sid * NC + cid
    _zero_acc(dacc, zbuf, sid, DW)
    ov = jnp.ones((16,), jnp.float32)

    def fill(r, carry):
        onesbuf[r, pl.ds(0, 16)] = ov
        return carry

    lax.fori_loop(0, CH, fill, 0)
    pltpu.sync_copy(edges_hbm.at[pl.ds(CBASE + wid * NCH, NCH)], colall)
    plsc.subcore_barrier()

    def body(step, carry):
        pltpu.sync_copy(onesbuf, dacc.at[colall.at[step]], add=True)
        return carry

    lax.fori_loop(0, NCH, body, 0)
    plsc.subcore_barrier()
    _writeback(dacc, out0, out1, cid, sid)


@functools.partial(
    pl.kernel,
    mesh=_mesh,
    compiler_params=_sc_params,
    out_type=[jax.ShapeDtypeStruct((NP, C), jnp.float32)] * 2,
    scratch_types=[
        pltpu.VMEM((NCH, CH), jnp.int32),
        pltpu.VMEM((NCH, CH), jnp.int32),
        pltpu.VMEM((NSLOT, CH, C), jnp.float32),
        pltpu.VMEM((PCH, C), jnp.float32),
        pltpu.VMEM_SHARED((NP, C), jnp.float32),
        [pltpu.SemaphoreType.DMA] * NSLOT,
        [pltpu.SemaphoreType.DMA] * NSLOT,
    ],
)
def _sc_hop(u_hbm, edges_hbm, out0, out1,
            rowall, colall, rows, zbuf, acc, semg, sems):
    cid = lax.axis_index("c")
    sid = lax.axis_index("s")
    wid = sid * NC + cid
    _zero_acc(acc, zbuf, sid, C)
    pltpu.sync_copy(edges_hbm.at[pl.ds(wid * NCH, NCH)], rowall)
    pltpu.sync_copy(edges_hbm.at[pl.ds(CBASE + wid * NCH, NCH)], colall)
    plsc.subcore_barrier()
    _edge_phase(u_hbm, acc, rowall, colall, rows, semg, sems)
    plsc.subcore_barrier()
    _writeback(acc, out0, out1, cid, sid)


def _tc_matmul_body(x_ref, w_ref, z_ref):
    z_ref[...] = lax.dot_general(x_ref[...], w_ref[...],
                                 (((1,), (1,)), ((), ())),
                                 preferred_element_type=jnp.float32)


_tc_matmul = pl.pallas_call(
    _tc_matmul_body,
    grid=(GT,),
    in_specs=[
        pl.BlockSpec((BT, D), lambda i: (i, 0)),
        pl.BlockSpec((C, D), lambda i: (0, 0)),
    ],
    out_specs=pl.BlockSpec((BT, C), lambda i: (i, 0)),
    out_shape=jax.ShapeDtypeStruct((NP, C), jnp.float32),
)


def _tc_scale_body(z_ref, d0_ref, d1_ref, u0_ref, scl_ref):
    deg = d0_ref[...][:, :1] + d1_ref[...][:, :1] + 2.0
    dis = lax.rsqrt(deg)
    dinv2 = 2.0 / deg
    u0_ref[...] = z_ref[...] * dis
    lanes = lax.broadcasted_iota(jnp.int32, (BT, DW), 1)
    scl_ref[...] = jnp.where(lanes == 0, dis, 0.0) + \
        jnp.where(lanes == 1, dinv2, 0.0)


_tc_scale = pl.pallas_call(
    _tc_scale_body,
    grid=(GT,),
    in_specs=[
        pl.BlockSpec((BT, C), lambda i: (i, 0)),
        pl.BlockSpec((BT, DW), lambda i: (i, 0)),
        pl.BlockSpec((BT, DW), lambda i: (i, 0)),
    ],
    out_specs=[
        pl.BlockSpec((BT, C), lambda i: (i, 0)),
        pl.BlockSpec((BT, DW), lambda i: (i, 0)),
    ],
    out_shape=[
        jax.ShapeDtypeStruct((NP, C), jnp.float32),
        jax.ShapeDtypeStruct((NP, DW), jnp.float32),
    ],
)


@functools.partial(
    pl.kernel,
    mesh=_mesh,
    compiler_params=_sc_params,
    out_type=[jax.ShapeDtypeStruct((NP, C), jnp.float32)] * 2,
    scratch_types=[
        pltpu.VMEM((RPW, C), jnp.float32),
        pltpu.VMEM((RPW, C), jnp.float32),
        pltpu.VMEM((RPW, C), jnp.float32),
        pltpu.VMEM((RPW, DW), jnp.float32),
        pltpu.VMEM((RPW, C), jnp.float32),
        pltpu.VMEM((RPW, C), jnp.float32),
    ],
)
def _sc_comb(p0_hbm, p1_hbm, z_hbm, scl_hbm, u1o, h1o,
             p0b, p1b, zb, sclb, u1b, h1b):
    cid = lax.axis_index("c")
    sid = lax.axis_index("s")
    wid = sid * NC + cid
    base = wid * RPW
    pltpu.sync_copy(p0_hbm.at[pl.ds(base, RPW)], p0b)
    pltpu.sync_copy(p1_hbm.at[pl.ds(base, RPW)], p1b)
    pltpu.sync_copy(z_hbm.at[pl.ds(base, RPW)], zb)
    pltpu.sync_copy(scl_hbm.at[pl.ds(base, RPW)], sclb)

    def rowfn(r, carry):
        sv = sclb[r, pl.ds(0, 16)]
        d = sv[0]
        v = sv[1]
        for j in range(C // 16):
            sl = pl.ds(j * 16, 16)
            h = d * (p0b[r, sl] + p1b[r, sl]) + v * zb[r, sl]
            h1b[r, sl] = h
            u1b[r, sl] = d * h
        return carry

    lax.fori_loop(0, RPW, rowfn, 0)
    pltpu.sync_copy(u1b, u1o.at[pl.ds(base, RPW)])
    pltpu.sync_copy(h1b, h1o.at[pl.ds(base, RPW)])


def _tc_comb2_body(p0_ref, p1_ref, scl_ref, h_ref, b_ref, o_ref):
    s = scl_ref[...]
    dis = s[:, :1]
    dinv2 = s[:, 1:2]
    t = dis * (p0_ref[...] + p1_ref[...]) + dinv2 * h_ref[...] + b_ref[...]
    m = jnp.max(t, axis=1, keepdims=True)
    lse = jnp.log(jnp.sum(jnp.exp(t - m), axis=1, keepdims=True)) + m
    o_ref[...] = t - lse


_tc_comb2 = pl.pallas_call(
    _tc_comb2_body,
    grid=(GT,),
    in_specs=[
        pl.BlockSpec((BT, C), lambda i: (i, 0)),
        pl.BlockSpec((BT, C), lambda i: (i, 0)),
        pl.BlockSpec((BT, DW), lambda i: (i, 0)),
        pl.BlockSpec((BT, C), lambda i: (i, 0)),
        pl.BlockSpec((1, C), lambda i: (0, 0)),
    ],
    out_specs=pl.BlockSpec((BT, C), lambda i: (i, 0)),
    out_shape=jax.ShapeDtypeStruct((N, C), jnp.float32),
)


def kernel(x, edge_index, W, b):
    edges = edge_index.reshape(2 * NW * NCH, CH)
    dg0, dg1 = _sc_deg(edges)
    z = _tc_matmul(x, W)
    u0, scl = _tc_scale(z, dg0, dg1)
    p10, p11 = _sc_hop(u0, edges)
    u1, h1 = _sc_comb(p10, p11, z, scl)
    p20, p21 = _sc_hop(u1, edges)
    return _tc_comb2(p20, p21, scl, h1, b.reshape(1, C))
